# baseline (speedup 1.0000x reference)
import functools

import jax
import jax.numpy as jnp
from jax import lax
from jax.experimental import pallas as pl
from jax.experimental.pallas import tpu as pltpu
from jax.experimental.pallas import tpu_sc as plsc

_LANES = 16


def _sc_segment_sums(x, ed_idx, *, chunk_rows=256):
    n, d = x.shape
    b = ed_idx.shape[0]
    nc, ns = 2, 16
    nw = nc * ns
    assert b % nw == 0 and d % _LANES == 0
    spw = b // nw
    assert spw == _LANES
    dv = d // _LANES

    mesh = plsc.VectorSubcoreMesh(core_axis_name="c", subcore_axis_name="s")

    @functools.partial(
        pl.kernel,
        out_type=jax.ShapeDtypeStruct((b, d), jnp.float32),
        mesh=mesh,
        scratch_types=[
            pltpu.VMEM((b,), jnp.int32),
            pltpu.VMEM((2, chunk_rows, d), jnp.float32),
            pltpu.VMEM((b // nw, d), jnp.float32),
            pltpu.SemaphoreType.DMA((2,)),
        ],
    )
    def seg_sum_kernel(x_hbm, ed_hbm, out_hbm, ed_v, buf, rows_v, sems):
        wid = lax.axis_index("s") * nc + lax.axis_index("c")
        pltpu.sync_copy(ed_hbm, ed_v)

        grp = ed_v[pl.ds(wid * spw, spw)]
        pg = ed_v[pl.ds(jnp.maximum(wid - 1, 0) * spw, spw)]
        prev = jnp.where(wid == 0, 0, pg[spw - 1])
        ends = [grp[k] for k in range(spw)]
        starts = [prev] + ends[:-1]

        zero = jnp.zeros((_LANES,), jnp.float32)
        for k in range(spw):
            for t in range(dv):
                rows_v[k, pl.ds(t * _LANES, _LANES)] = zero

        astart = (starts[0] // 8) * 8
        nch = lax.div(ends[-1] - astart + (chunk_rows - 1), chunk_rows)

        def chunk_dma(c):
            base = jnp.minimum(astart + c * chunk_rows, n - chunk_rows)
            return pltpu.make_async_copy(
                x_hbm.at[pl.ds(base, chunk_rows)], buf.at[c % 2], sems.at[c % 2]
            )

        @pl.when(nch > 0)
        def _():
            chunk_dma(0).start()

        def chunk_body(c, _):
            @pl.when(c + 1 < nch)
            def _():
                chunk_dma(c + 1).start()

            chunk_dma(c).wait()
            cb = astart + c * chunk_rows
            base = jnp.minimum(cb, n - chunk_rows)
            p = c % 2
            for k in range(spw):
                lo = jnp.maximum(starts[k], cb) - base
                hi = jnp.minimum(ends[k], cb + chunk_rows) - base

                @pl.when(lo < hi)
                def _(lo=lo, hi=hi, k=k):
                    n4 = (hi - lo) // 4

                    def row4_body(i, accs):
                        r = lo + i * 4
                        for u in range(4):
                            accs = tuple(
                                accs[t] + buf[p, r + u, pl.ds(t * _LANES, _LANES)]
                                for t in range(dv)
                            )
                        return accs

                    def row_body(r, accs):
                        return tuple(
                            accs[t] + buf[p, r, pl.ds(t * _LANES, _LANES)]
                            for t in range(dv)
                        )

                    accs = lax.fori_loop(0, n4, row4_body, (zero,) * dv)
                    accs = lax.fori_loop(lo + n4 * 4, hi, row_body, accs)
                    for t in range(dv):
                        sl = pl.ds(t * _LANES, _LANES)
                        rows_v[k, sl] = rows_v[k, sl] + accs[t]

            return 0

        lax.fori_loop(0, nch, chunk_body, 0)
        pltpu.sync_copy(rows_v, out_hbm.at[pl.ds(wid * spw, spw)])

    return seg_sum_kernel(x, ed_idx)


def _tc_tail_kernel(sums_ref, inv_counts_ref, clsT_ref, bias_ref, out_ref):
    ave = sums_ref[...] * inv_counts_ref[...]
    t = ave[:, 0:1]
    inner = jnp.sum(ave * ave, axis=1, keepdims=True) - 2.0 * t * t
    denom = jnp.sqrt(jnp.maximum(jnp.abs(inner), 1e-8))
    cx = ave / denom
    col = lax.broadcasted_iota(jnp.int32, cx.shape, 1)
    cx = jnp.where(col == 0, -cx, cx)
    logits = jnp.dot(cx, clsT_ref[...], preferred_element_type=jnp.float32)
    out_ref[...] = 2.0 + 2.0 * logits + bias_ref[...]


def kernel(x, ed_idx, cls, bias):
    b = ed_idx.shape[0]
    c = cls.shape[0]
    sums = _sc_segment_sums(x, ed_idx)
    starts = jnp.concatenate([jnp.zeros((1,), ed_idx.dtype), ed_idx[:-1]])
    counts = jnp.maximum((ed_idx - starts).astype(jnp.float32), 1.0)
    inv_counts = (1.0 / counts)[:, None]
    out = pl.pallas_call(
        _tc_tail_kernel,
        out_shape=jax.ShapeDtypeStruct((b, c), jnp.float32),
    )(sums, inv_counts, cls.T, bias[None, :])
    return out


# speedup vs baseline: 2.2244x; 2.2244x over previous
import functools

import jax
import jax.numpy as jnp
from jax import lax
from jax.experimental import pallas as pl
from jax.experimental.pallas import tpu as pltpu
from jax.experimental.pallas import tpu_sc as plsc

_LANES = 16
_L2 = 64


def _tc_block_sums(x, *, rows_per_step=12800):
    n, d = x.shape
    nb = rows_per_step // _L2

    def blocksum_kernel(x_ref, out_ref):
        tile = x_ref[...]
        out_ref[...] = jnp.sum(tile.reshape(nb, _L2, d), axis=1)

    return pl.pallas_call(
        blocksum_kernel,
        grid=(n // rows_per_step,),
        in_specs=[pl.BlockSpec((rows_per_step, d), lambda i: (i, 0))],
        out_specs=pl.BlockSpec((nb, d), lambda i: (i, 0)),
        out_shape=jax.ShapeDtypeStruct((n // _L2, d), jnp.float32),
    )(x)


def _sc_boundary_sums(x, ed_idx):
    n, d = x.shape
    b = ed_idx.shape[0]
    nc, ns = 2, 16
    nw = nc * ns
    assert b % nw == 0 and d % _LANES == 0
    spw = b // nw
    assert spw == _LANES
    dv = d // _LANES

    mesh = plsc.VectorSubcoreMesh(core_axis_name="c", subcore_axis_name="s")

    @functools.partial(
        pl.kernel,
        out_type=jax.ShapeDtypeStruct((b, d), jnp.float32),
        mesh=mesh,
        scratch_types=[
            pltpu.VMEM((b,), jnp.int32),
            pltpu.VMEM((2, _L2, d), jnp.float32),
            pltpu.VMEM((spw, d), jnp.float32),
            pltpu.SemaphoreType.DMA((2,)),
        ],
    )
    def boundary_kernel(x_hbm, ed_hbm, out_hbm, ed_v, buf, rows_v, sems):
        wid = lax.axis_index("s") * nc + lax.axis_index("c")
        pltpu.sync_copy(ed_hbm, ed_v)

        grp = ed_v[pl.ds(wid * spw, spw)]
        es = [grp[k] for k in range(spw)]
        zero = jnp.zeros((_LANES,), jnp.float32)

        def win_dma(e, p):
            wstart = (e // _L2) * _L2
            return pltpu.make_async_copy(
                x_hbm.at[pl.ds(wstart, _L2)], buf.at[p], sems.at[p]
            )

        win_dma(es[0], 0).start()
        for k in range(spw):
            p = k % 2
            if k + 1 < spw:
                win_dma(es[k + 1], (k + 1) % 2).start()
            win_dma(es[k], p).wait()
            m = es[k] % _L2

            def row_body(r, accs, p=p):
                return tuple(
                    accs[t] + buf[p, r, pl.ds(t * _LANES, _LANES)]
                    for t in range(dv)
                )

            accs = lax.fori_loop(0, m, row_body, (zero,) * dv)
            for t in range(dv):
                rows_v[k, pl.ds(t * _LANES, _LANES)] = accs[t]
        pltpu.sync_copy(rows_v, out_hbm.at[pl.ds(wid * spw, spw)])

    return boundary_kernel(x, ed_idx)


def _tc_tail_kernel(sb_ref, t_ref, bs_ref, be_ref, invc_ref, clsT_ref,
                    bias_ref, out_ref):
    kk = sb_ref.shape[0]
    bb = t_ref.shape[0]
    kio = lax.broadcasted_iota(jnp.int32, (bb, kk), 1)
    msel = ((kio >= bs_ref[...]) & (kio < be_ref[...])).astype(jnp.float32)
    sums = jnp.dot(msel, sb_ref[...], preferred_element_type=jnp.float32)
    tcur = t_ref[...]
    rio = lax.broadcasted_iota(jnp.int32, tcur.shape, 0)
    tprev = jnp.where(rio == 0, 0.0, pltpu.roll(tcur, 1, 0))
    sums = sums + tcur - tprev
    ave = sums * invc_ref[...]
    t0 = ave[:, 0:1]
    inner = jnp.sum(ave * ave, axis=1, keepdims=True) - 2.0 * t0 * t0
    denom = jnp.sqrt(jnp.maximum(jnp.abs(inner), 1e-8))
    cx = ave / denom
    col = lax.broadcasted_iota(jnp.int32, cx.shape, 1)
    cx = jnp.where(col == 0, -cx, cx)
    logits = jnp.dot(cx, clsT_ref[...], preferred_element_type=jnp.float32)
    out_ref[...] = 2.0 + 2.0 * logits + bias_ref[...]


def kernel(x, ed_idx, cls, bias):
    b = ed_idx.shape[0]
    c = cls.shape[0]
    sb = _tc_block_sums(x)
    t = _sc_boundary_sums(x, ed_idx)
    starts = jnp.concatenate([jnp.zeros((1,), ed_idx.dtype), ed_idx[:-1]])
    counts = jnp.maximum((ed_idx - starts).astype(jnp.float32), 1.0)
    inv_counts = (1.0 / counts)[:, None]
    bs2 = (starts // _L2)[:, None]
    be2 = (ed_idx // _L2)[:, None]
    out = pl.pallas_call(
        _tc_tail_kernel,
        out_shape=jax.ShapeDtypeStruct((b, c), jnp.float32),
    )(sb, t, bs2, be2, inv_counts, cls.T, bias[None, :])
    return out
